# Initial kernel scaffold; baseline (speedup 1.0000x reference)
import jax
import jax.numpy as jnp
from jax.experimental import pallas as pl


def kernel(etab, E_idx, ref_seqs, x_mask):
    raise NotImplementedError("write your pallas kernel here")


# speedup vs baseline: 1.4149x; 1.4149x over previous
import functools

import jax
import jax.numpy as jnp
from jax import lax
from jax.experimental import pallas as pl
from jax.experimental.pallas import tpu as pltpu
from jax.experimental.pallas import tpu_sc as plsc

B, L, K, AA = 8, 512, 30, 20
S = B * L
BLK = AA * AA

NC, NS, LANES = 2, 16, 16
NW = NC * NS
SPW = S // NW
CH = 4
NCHUNK = SPW // CH
CHW = CH * K * BLK
BUF_W = CHW + 256


def _sc_body(etab_hbm, eidx_hbm, ref_hbm, out_hbm,
             buf0, buf1, eidx_v, ref_v, out_v, sem0, sem1):
    wid = lax.axis_index("s") * NC + lax.axis_index("c")
    wbase = wid * SPW
    b512 = (wid // (L // SPW)) * L

    pltpu.sync_copy(eidx_hbm.at[pl.ds(wbase * K, SPW * K)], eidx_v)
    pltpu.sync_copy(ref_hbm, ref_v)

    i_lo = lax.iota(jnp.int32, LANES)
    i_hi = i_lo + LANES
    hi_mask = i_lo < (AA - LANES)
    hi_sc = jnp.minimum(i_lo, AA - LANES - 1) + LANES

    def chunk_src(c):
        return etab_hbm.at[pl.ds((wbase + c * CH) * K * BLK, CHW)]

    def process(c, buf):
        for sl in range(CH):
            site = c * CH + sl
            bb = sl * K * BLK
            acc0 = plsc.load_gather(buf, [bb + i_lo * (AA + 1)])
            acc1 = plsc.load_gather(buf, [bb + i_hi * (AA + 1)])

            def kbody(k, accs):
                a0, a1 = accs
                eidx = plsc.load_gather(
                    eidx_v, [jnp.full((LANES,), site * K + k, jnp.int32)])
                eaa = plsc.load_gather(ref_v, [eidx + b512])
                kb = bb + k * BLK
                a0 = a0 + plsc.load_gather(buf, [kb + i_lo * AA + eaa])
                a1 = a1 + plsc.load_gather(buf, [kb + i_hi * AA + eaa])
                return a0, a1

            acc0, acc1 = lax.fori_loop(1, K, kbody, (acc0, acc1))
            ob = site * AA
            out_v[pl.ds(ob, LANES)] = acc0
            plsc.store_scatter(out_v, [ob + hi_sc], acc1, mask=hi_mask)

    pltpu.async_copy(chunk_src(0), buf0.at[pl.ds(0, CHW)], sem0)

    def gbody(g, _):
        c0 = 2 * g
        c1 = c0 + 1
        pltpu.async_copy(chunk_src(c1), buf1.at[pl.ds(0, CHW)], sem1)
        pltpu.make_async_copy(chunk_src(c0), buf0.at[pl.ds(0, CHW)], sem0).wait()
        process(c0, buf0)

        @pl.when(c1 + 1 < NCHUNK)
        def _():
            pltpu.async_copy(chunk_src(c1 + 1), buf0.at[pl.ds(0, CHW)], sem0)

        pltpu.make_async_copy(chunk_src(c1), buf1.at[pl.ds(0, CHW)], sem1).wait()
        process(c1, buf1)
        return 0

    lax.fori_loop(0, NCHUNK // 2, gbody, 0)

    pltpu.sync_copy(out_v, out_hbm.at[pl.ds(wbase * AA, SPW * AA)])


_SC_GATHER_CACHE = []


def _sc_gather(*args):
    if not _SC_GATHER_CACHE:
        _SC_GATHER_CACHE.append(functools.partial(
            pl.kernel,
            out_type=jax.ShapeDtypeStruct((S * AA,), jnp.float32),
            mesh=plsc.VectorSubcoreMesh(core_axis_name="c",
                                        subcore_axis_name="s",
                                        num_cores=NC, num_subcores=NS),
            scratch_types=[
                pltpu.VMEM((BUF_W,), jnp.float32),
                pltpu.VMEM((BUF_W,), jnp.float32),
                pltpu.VMEM((SPW * K,), jnp.int32),
                pltpu.VMEM((S,), jnp.int32),
                pltpu.VMEM((SPW * AA,), jnp.float32),
                pltpu.SemaphoreType.DMA,
                pltpu.SemaphoreType.DMA,
            ],
            compiler_params=pltpu.CompilerParams(needs_layout_passes=False),
        )(_sc_body))
    return _SC_GATHER_CACHE[0](*args)


def _fin_body(aa_ref, ref_ref, mask_ref, out_ref):
    neg = -aa_ref[...]
    m = jnp.max(neg, axis=-1, keepdims=True)
    lse = jnp.log(jnp.sum(jnp.exp(neg - m), axis=-1)) + m[..., 0]
    r = ref_ref[...]
    sel = lax.broadcasted_iota(jnp.int32, (B, L, AA), 2) == r[:, :, None]
    picked = jnp.sum(jnp.where(sel, neg, 0.0), axis=-1)
    mask = mask_ref[...]
    num = jnp.sum((picked - lse) * mask, axis=1, keepdims=True)
    den = jnp.sum(mask, axis=1, keepdims=True)
    out_ref[0, 0] = -jnp.sum(num / den) / B


_finish = pl.pallas_call(
    _fin_body,
    out_shape=jax.ShapeDtypeStruct((1, 1), jnp.float32),
    out_specs=pl.BlockSpec(memory_space=pltpu.SMEM),
)


def kernel(etab, E_idx, ref_seqs, x_mask):
    aa_nrgs = _sc_gather(
        etab.reshape(-1),
        E_idx.reshape(-1),
        ref_seqs.reshape(-1),
    )
    out = _finish(aa_nrgs.reshape(B, L, AA), ref_seqs, x_mask)
    return out[0, 0]


# speedup vs baseline: 2.2741x; 1.6072x over previous
import functools

import jax
import jax.numpy as jnp
from jax import lax
from jax.experimental import pallas as pl
from jax.experimental.pallas import tpu as pltpu
from jax.experimental.pallas import tpu_sc as plsc

B, L, K, AA = 8, 512, 30, 20
S = B * L
BLK = AA * AA

NC, NS, LANES = 2, 16, 16
NW = NC * NS
SPW = S // NW
CH = 4
NCHUNK = SPW // CH
CHW = CH * K * BLK
BUF_W = CHW + 256


def _sc_body(etab_hbm, eidx_hbm, ref_hbm, out_hbm,
             buf0, buf1, eidx_v, ref_v, out_v, sem0, sem1):
    wid = lax.axis_index("s") * NC + lax.axis_index("c")
    wbase = wid * SPW
    b512 = (wid // (L // SPW)) * L

    pltpu.sync_copy(eidx_hbm.at[pl.ds(wbase * K, SPW * K)], eidx_v)
    pltpu.sync_copy(ref_hbm, ref_v)

    i_lo = lax.iota(jnp.int32, LANES)
    i_hi = i_lo + LANES
    hi_mask = i_lo < (AA - LANES)
    hi_sc = jnp.minimum(i_lo, AA - LANES - 1) + LANES

    def splat(x):
        return jnp.full((LANES,), x, jnp.int32)

    def chunk_src(c):
        return etab_hbm.at[wbase + c]

    def process(c, buf):
        site = c
        acc0 = plsc.load_gather(buf, [splat(0), i_lo * (AA + 1)])
        acc1 = plsc.load_gather(
            buf, [splat(0), jnp.minimum(i_hi * (AA + 1), BLK - 1)])

        def kbody(k, accs):
            a0, a1 = accs
            eidx = plsc.load_gather(eidx_v, [splat(site * K + k)])
            eaa = plsc.load_gather(ref_v, [eidx + b512])
            kk = splat(k)
            a0 = a0 + plsc.load_gather(buf, [kk, i_lo * AA + eaa])
            a1 = a1 + plsc.load_gather(
                buf, [kk, jnp.minimum(i_hi * AA + eaa, BLK - 1)])
            return a0, a1

        acc0, acc1 = lax.fori_loop(1, K, kbody, (acc0, acc1))
        ob = site * AA
        out_v[pl.ds(ob, LANES)] = acc0
        plsc.store_scatter(out_v, [ob + hi_sc], acc1, mask=hi_mask)

    pltpu.async_copy(chunk_src(0), buf0, sem0)

    def gbody(g, _):
        c0 = 2 * g
        c1 = c0 + 1
        pltpu.async_copy(chunk_src(c1), buf1, sem1)
        pltpu.make_async_copy(chunk_src(c0), buf0, sem0).wait()
        process(c0, buf0)

        @pl.when(c1 + 1 < SPW)
        def _():
            pltpu.async_copy(chunk_src(c1 + 1), buf0, sem0)

        pltpu.make_async_copy(chunk_src(c1), buf1, sem1).wait()
        process(c1, buf1)
        return 0

    lax.fori_loop(0, SPW // 2, gbody, 0)

    pltpu.sync_copy(out_v, out_hbm.at[pl.ds(wbase * AA, SPW * AA)])


_SC_GATHER_CACHE = []


def _sc_gather(*args):
    if not _SC_GATHER_CACHE:
        _SC_GATHER_CACHE.append(functools.partial(
            pl.kernel,
            out_type=jax.ShapeDtypeStruct((S * AA,), jnp.float32),
            mesh=plsc.VectorSubcoreMesh(core_axis_name="c",
                                        subcore_axis_name="s",
                                        num_cores=NC, num_subcores=NS),
            scratch_types=[
                pltpu.VMEM((K, BLK), jnp.float32),
                pltpu.VMEM((K, BLK), jnp.float32),
                pltpu.VMEM((SPW * K,), jnp.int32),
                pltpu.VMEM((S,), jnp.int32),
                pltpu.VMEM((SPW * AA,), jnp.float32),
                pltpu.SemaphoreType.DMA,
                pltpu.SemaphoreType.DMA,
            ],
            compiler_params=pltpu.CompilerParams(needs_layout_passes=False),
        )(_sc_body))
    return _SC_GATHER_CACHE[0](*args)


def _fin_body(aa_ref, ref_ref, mask_ref, out_ref):
    neg = -aa_ref[...]
    m = jnp.max(neg, axis=-1, keepdims=True)
    lse = jnp.log(jnp.sum(jnp.exp(neg - m), axis=-1)) + m[..., 0]
    r = ref_ref[...]
    sel = lax.broadcasted_iota(jnp.int32, (B, L, AA), 2) == r[:, :, None]
    picked = jnp.sum(jnp.where(sel, neg, 0.0), axis=-1)
    mask = mask_ref[...]
    num = jnp.sum((picked - lse) * mask, axis=1, keepdims=True)
    den = jnp.sum(mask, axis=1, keepdims=True)
    out_ref[0, 0] = -jnp.sum(num / den) / B


_finish = pl.pallas_call(
    _fin_body,
    out_shape=jax.ShapeDtypeStruct((1, 1), jnp.float32),
    out_specs=pl.BlockSpec(memory_space=pltpu.SMEM),
)


def kernel(etab, E_idx, ref_seqs, x_mask):
    aa_nrgs = _sc_gather(
        etab.reshape(S, K, BLK),
        E_idx.reshape(-1),
        ref_seqs.reshape(-1),
    )
    out = _finish(aa_nrgs.reshape(B, L, AA), ref_seqs, x_mask)
    return out[0, 0]


# speedup vs baseline: 2.2755x; 1.0006x over previous
import functools

import jax
import jax.numpy as jnp
from jax import lax
from jax.experimental import pallas as pl
from jax.experimental.pallas import tpu as pltpu
from jax.experimental.pallas import tpu_sc as plsc

B, L, K, AA = 8, 512, 30, 20
S = B * L
BLK = AA * AA

NC, NS, LANES = 2, 16, 16
NW = NC * NS
SPW = S // NW
CH = 4
NCHUNK = SPW // CH
CHW = CH * K * BLK
BUF_W = CHW + 256


def _sc_body(etab_hbm, eidx_hbm, ref_hbm, out_hbm,
             buf0, buf1, eidx_v, ref_v, out_v, sem0, sem1):
    wid = lax.axis_index("s") * NC + lax.axis_index("c")
    wbase = wid * SPW
    b512 = (wid // (L // SPW)) * L

    pltpu.sync_copy(eidx_hbm.at[pl.ds(wbase * K, SPW * K)], eidx_v)
    pltpu.sync_copy(ref_hbm, ref_v)

    i_lo = lax.iota(jnp.int32, LANES)
    i_hi = i_lo + LANES
    hi_mask = i_lo < (AA - LANES)
    hi_sc = jnp.minimum(i_lo, AA - LANES - 1) + LANES

    def splat(x):
        return jnp.full((LANES,), x, jnp.int32)

    def chunk_src(c):
        return etab_hbm.at[wbase + c]

    def process(c, buf):
        site = c
        acc0 = plsc.load_gather(buf, [splat(0), i_lo * (AA + 1)])
        acc1 = plsc.load_gather(
            buf, [splat(0), jnp.minimum(i_hi * (AA + 1), BLK - 1)])

        def kbody(k, accs):
            a0, a1 = accs
            eidx = plsc.load_gather(eidx_v, [splat(site * K + k)])
            eaa = plsc.load_gather(ref_v, [eidx + b512])
            kk = splat(k)
            a0 = a0 + plsc.load_gather(buf, [kk, i_lo * AA + eaa])
            a1 = a1 + plsc.load_gather(
                buf, [kk, jnp.minimum(i_hi * AA + eaa, BLK - 1)])
            return a0, a1

        acc0, acc1 = lax.fori_loop(1, K, kbody, (acc0, acc1))
        ob = site * AA
        out_v[pl.ds(ob, LANES)] = acc0
        plsc.store_scatter(out_v, [ob + hi_sc], acc1, mask=hi_mask)

    pltpu.async_copy(chunk_src(0), buf0, sem0)

    def gbody(g, _):
        c0 = 2 * g
        c1 = c0 + 1
        pltpu.async_copy(chunk_src(c1), buf1, sem1)
        pltpu.make_async_copy(chunk_src(c0), buf0, sem0).wait()
        process(c0, buf0)

        @pl.when(c1 + 1 < SPW)
        def _():
            pltpu.async_copy(chunk_src(c1 + 1), buf0, sem0)

        pltpu.make_async_copy(chunk_src(c1), buf1, sem1).wait()
        process(c1, buf1)
        return 0

    lax.fori_loop(0, SPW // 2, gbody, 0)

    pltpu.sync_copy(out_v, out_hbm.at[pl.ds(wbase * AA, SPW * AA)])


_SC_GATHER_CACHE = []


def _sc_gather(*args):
    if not _SC_GATHER_CACHE:
        _SC_GATHER_CACHE.append(functools.partial(
            pl.kernel,
            out_type=jax.ShapeDtypeStruct((S * AA,), jnp.float32),
            mesh=plsc.VectorSubcoreMesh(core_axis_name="c",
                                        subcore_axis_name="s",
                                        num_cores=NC, num_subcores=NS),
            scratch_types=[
                pltpu.VMEM((K, BLK), jnp.float32),
                pltpu.VMEM((K, BLK), jnp.float32),
                pltpu.VMEM((SPW * K,), jnp.int32),
                pltpu.VMEM((S,), jnp.int32),
                pltpu.VMEM((SPW * AA,), jnp.float32),
                pltpu.SemaphoreType.DMA,
                pltpu.SemaphoreType.DMA,
            ],
            compiler_params=pltpu.CompilerParams(needs_layout_passes=False,
                                                 use_tc_tiling_on_sc=True),
        )(_sc_body))
    return _SC_GATHER_CACHE[0](*args)


def _fin_body(aa_ref, ref_ref, mask_ref, out_ref):
    neg = -aa_ref[...]
    m = jnp.max(neg, axis=-1, keepdims=True)
    lse = jnp.log(jnp.sum(jnp.exp(neg - m), axis=-1)) + m[..., 0]
    r = ref_ref[...]
    sel = lax.broadcasted_iota(jnp.int32, (B, L, AA), 2) == r[:, :, None]
    picked = jnp.sum(jnp.where(sel, neg, 0.0), axis=-1)
    mask = mask_ref[...]
    num = jnp.sum((picked - lse) * mask, axis=1, keepdims=True)
    den = jnp.sum(mask, axis=1, keepdims=True)
    out_ref[0, 0] = -jnp.sum(num / den) / B


_finish = pl.pallas_call(
    _fin_body,
    out_shape=jax.ShapeDtypeStruct((1, 1), jnp.float32),
    out_specs=pl.BlockSpec(memory_space=pltpu.SMEM),
)


def kernel(etab, E_idx, ref_seqs, x_mask):
    aa_nrgs = _sc_gather(
        etab.reshape(S, K, BLK),
        E_idx.reshape(-1),
        ref_seqs.reshape(-1),
    )
    out = _finish(aa_nrgs.reshape(B, L, AA), ref_seqs, x_mask)
    return out[0, 0]


# speedup vs baseline: 2.4085x; 1.0585x over previous
import functools

import jax
import jax.numpy as jnp
from jax import lax
from jax.experimental import pallas as pl
from jax.experimental.pallas import tpu as pltpu
from jax.experimental.pallas import tpu_sc as plsc

B, L, K, AA = 8, 512, 30, 20
S = B * L
BLK = AA * AA

NC, NS, LANES = 2, 16, 16
NW = NC * NS
SPW = S // NW
CH = 4
NCHUNK = SPW // CH
CHW = CH * K * BLK
BUF_W = CHW + 256


def _sc_body(etab_hbm, eidx_hbm, ref_hbm, out_hbm,
             buf0, buf1, eidx_v, ref_v, out_v, sem0, sem1):
    wid = lax.axis_index("s") * NC + lax.axis_index("c")
    wbase = wid * SPW
    b512 = (wid // (L // SPW)) * L

    pltpu.sync_copy(eidx_hbm.at[pl.ds(wbase * K, SPW * K)], eidx_v)
    pltpu.sync_copy(ref_hbm, ref_v)

    i_lo = lax.iota(jnp.int32, LANES)
    i_hi = i_lo + LANES
    hi_mask = i_lo < (AA - LANES)
    hi_sc = jnp.minimum(i_lo, AA - LANES - 1) + LANES

    def splat(x):
        return jnp.full((LANES,), x, jnp.int32)

    def chunk_src(c):
        return etab_hbm.at[wbase + c]

    def process(c, buf):
        site = c
        acc0 = plsc.load_gather(buf, [splat(0), i_lo * (AA + 1)])
        acc1 = plsc.load_gather(
            buf, [splat(0), jnp.minimum(i_hi * (AA + 1), BLK - 1)])

        def kbody(k, accs):
            a0, a1 = accs
            eidx = plsc.load_gather(eidx_v, [splat(site * K + k)])
            eaa = plsc.load_gather(ref_v, [eidx + b512])
            kk = splat(k)
            a0 = a0 + plsc.load_gather(buf, [kk, i_lo * AA + eaa])
            a1 = a1 + plsc.load_gather(
                buf, [kk, jnp.minimum(i_hi * AA + eaa, BLK - 1)])
            return a0, a1

        acc0, acc1 = lax.fori_loop(1, K, kbody, (acc0, acc1))
        ob = site * AA
        out_v[pl.ds(ob, LANES)] = acc0
        plsc.store_scatter(out_v, [ob + hi_sc], acc1, mask=hi_mask)

    pltpu.async_copy(chunk_src(0), buf0, sem0)

    def gbody(g, _):
        c0 = 2 * g
        c1 = c0 + 1
        pltpu.async_copy(chunk_src(c1), buf1, sem1)
        pltpu.make_async_copy(chunk_src(c0), buf0, sem0).wait()
        process(c0, buf0)

        @pl.when(c1 + 1 < SPW)
        def _():
            pltpu.async_copy(chunk_src(c1 + 1), buf0, sem0)

        pltpu.make_async_copy(chunk_src(c1), buf1, sem1).wait()
        process(c1, buf1)
        return 0

    lax.fori_loop(0, SPW // 2, gbody, 0)

    pltpu.sync_copy(out_v, out_hbm.at[pl.ds(wbase * AA, SPW * AA)])


_SC_GATHER_CACHE = []


def _sc_gather(*args):
    if not _SC_GATHER_CACHE:
        _SC_GATHER_CACHE.append(functools.partial(
            pl.kernel,
            out_type=jax.ShapeDtypeStruct((S * AA,), jnp.float32),
            mesh=plsc.VectorSubcoreMesh(core_axis_name="c",
                                        subcore_axis_name="s",
                                        num_cores=NC, num_subcores=NS),
            scratch_types=[
                pltpu.VMEM((K, BLK), jnp.float32),
                pltpu.VMEM((K, BLK), jnp.float32),
                pltpu.VMEM((SPW * K,), jnp.int32),
                pltpu.VMEM((S,), jnp.int32),
                pltpu.VMEM((SPW * AA,), jnp.float32),
                pltpu.SemaphoreType.DMA,
                pltpu.SemaphoreType.DMA,
            ],
            compiler_params=pltpu.CompilerParams(needs_layout_passes=False,
                                                 use_tc_tiling_on_sc=True),
        )(_sc_body))
    return _SC_GATHER_CACHE[0](*args)


def _eaa_body(eidx_hbm, ref_hbm, out_hbm, eidx_v, ref_v, out_v):
    wid = lax.axis_index("s") * NC + lax.axis_index("c")
    wbase = wid * SPW
    b512 = (wid // (L // SPW)) * L
    pltpu.sync_copy(eidx_hbm.at[pl.ds(wbase * K, SPW * K)], eidx_v)
    pltpu.sync_copy(ref_hbm, ref_v)

    def body(g, _):
        eidx = eidx_v[pl.ds(g * LANES, LANES)]
        out_v[pl.ds(g * LANES, LANES)] = plsc.load_gather(ref_v, [eidx + b512])
        return 0

    lax.fori_loop(0, SPW * K // LANES, body, 0)
    pltpu.sync_copy(out_v, out_hbm.at[pl.ds(wbase * K, SPW * K)])


_EAA_CACHE = []


def _sc_eaa(*args):
    if not _EAA_CACHE:
        _EAA_CACHE.append(functools.partial(
            pl.kernel,
            out_type=jax.ShapeDtypeStruct((S * K,), jnp.int32),
            mesh=plsc.VectorSubcoreMesh(core_axis_name="c",
                                        subcore_axis_name="s",
                                        num_cores=NC, num_subcores=NS),
            scratch_types=[
                pltpu.VMEM((SPW * K,), jnp.int32),
                pltpu.VMEM((S,), jnp.int32),
                pltpu.VMEM((SPW * K,), jnp.int32),
            ],
            compiler_params=pltpu.CompilerParams(needs_layout_passes=False),
        )(_eaa_body))
    return _EAA_CACHE[0](*args)


SB = 64

_SEG = None


def _tc_body(eaa_ref, etab_ref, out_ref):
    e = etab_ref[...]
    eaa = eaa_ref[...]
    j = lax.broadcasted_iota(jnp.int32, (SB, K, BLK), 2)
    jm = j % AA
    jd = j // AA
    kk = lax.broadcasted_iota(jnp.int32, (SB, K, BLK), 1)
    is0 = kk == 0
    cond = (is0 & (jm == jd)) | (~is0 & (jm == eaa[:, :, None]))
    msum = jnp.sum(jnp.where(cond, e, 0.0), axis=1)
    seg = (lax.broadcasted_iota(jnp.int32, (BLK, AA), 0) // AA ==
           lax.broadcasted_iota(jnp.int32, (BLK, AA), 1)).astype(jnp.float32)
    out_ref[...] = jnp.dot(msum, seg, preferred_element_type=jnp.float32)


def _tc_gather(eaa, etab3, n_sites):
    return pl.pallas_call(
        _tc_body,
        grid=(n_sites // SB,),
        in_specs=[
            pl.BlockSpec((SB, K), lambda i: (i, 0)),
            pl.BlockSpec((SB, K, BLK), lambda i: (i, 0, 0)),
        ],
        out_specs=pl.BlockSpec((SB, AA), lambda i: (i, 0)),
        out_shape=jax.ShapeDtypeStruct((n_sites, AA), jnp.float32),
    )(eaa, etab3)


def _fin_body(aa_ref, ref_ref, mask_ref, out_ref):
    neg = -aa_ref[...]
    m = jnp.max(neg, axis=-1, keepdims=True)
    lse = jnp.log(jnp.sum(jnp.exp(neg - m), axis=-1)) + m[..., 0]
    r = ref_ref[...]
    sel = lax.broadcasted_iota(jnp.int32, (B, L, AA), 2) == r[:, :, None]
    picked = jnp.sum(jnp.where(sel, neg, 0.0), axis=-1)
    mask = mask_ref[...]
    num = jnp.sum((picked - lse) * mask, axis=1, keepdims=True)
    den = jnp.sum(mask, axis=1, keepdims=True)
    out_ref[0, 0] = -jnp.sum(num / den) / B


_finish = pl.pallas_call(
    _fin_body,
    out_shape=jax.ShapeDtypeStruct((1, 1), jnp.float32),
    out_specs=pl.BlockSpec(memory_space=pltpu.SMEM),
)


def kernel(etab, E_idx, ref_seqs, x_mask):
    etab3 = etab.reshape(S, K, BLK)
    eaa = _sc_eaa(E_idx.reshape(-1), ref_seqs.reshape(-1))
    aa_nrgs = _tc_gather(eaa.reshape(S, K), etab3, S)
    out = _finish(aa_nrgs.reshape(B, L, AA), ref_seqs, x_mask)
    return out[0, 0]


# speedup vs baseline: 3.3168x; 1.3771x over previous
import functools

import jax
import jax.numpy as jnp
from jax import lax
from jax.experimental import pallas as pl
from jax.experimental.pallas import tpu as pltpu
from jax.experimental.pallas import tpu_sc as plsc

B, L, K, AA = 8, 512, 30, 20
S = B * L
BLK = AA * AA

NC, NS, LANES = 2, 16, 16
NW = NC * NS
SPW = S // NW
CH = 4
NCHUNK = SPW // CH
CHW = CH * K * BLK
BUF_W = CHW + 256


def _sc_body(etab_hbm, eidx_hbm, ref_hbm, out_hbm,
             buf0, buf1, eidx_v, ref_v, out_v, sem0, sem1):
    wid = lax.axis_index("s") * NC + lax.axis_index("c")
    wbase = wid * SPW
    b512 = (wid // (L // SPW)) * L

    pltpu.sync_copy(eidx_hbm.at[pl.ds(wbase * K, SPW * K)], eidx_v)
    pltpu.sync_copy(ref_hbm, ref_v)

    i_lo = lax.iota(jnp.int32, LANES)
    i_hi = i_lo + LANES
    hi_mask = i_lo < (AA - LANES)
    hi_sc = jnp.minimum(i_lo, AA - LANES - 1) + LANES

    def splat(x):
        return jnp.full((LANES,), x, jnp.int32)

    def chunk_src(c):
        return etab_hbm.at[wbase + c]

    def process(c, buf):
        site = c
        acc0 = plsc.load_gather(buf, [splat(0), i_lo * (AA + 1)])
        acc1 = plsc.load_gather(
            buf, [splat(0), jnp.minimum(i_hi * (AA + 1), BLK - 1)])

        def kbody(k, accs):
            a0, a1 = accs
            eidx = plsc.load_gather(eidx_v, [splat(site * K + k)])
            eaa = plsc.load_gather(ref_v, [eidx + b512])
            kk = splat(k)
            a0 = a0 + plsc.load_gather(buf, [kk, i_lo * AA + eaa])
            a1 = a1 + plsc.load_gather(
                buf, [kk, jnp.minimum(i_hi * AA + eaa, BLK - 1)])
            return a0, a1

        acc0, acc1 = lax.fori_loop(1, K, kbody, (acc0, acc1))
        ob = site * AA
        out_v[pl.ds(ob, LANES)] = acc0
        plsc.store_scatter(out_v, [ob + hi_sc], acc1, mask=hi_mask)

    pltpu.async_copy(chunk_src(0), buf0, sem0)

    def gbody(g, _):
        c0 = 2 * g
        c1 = c0 + 1
        pltpu.async_copy(chunk_src(c1), buf1, sem1)
        pltpu.make_async_copy(chunk_src(c0), buf0, sem0).wait()
        process(c0, buf0)

        @pl.when(c1 + 1 < SPW)
        def _():
            pltpu.async_copy(chunk_src(c1 + 1), buf0, sem0)

        pltpu.make_async_copy(chunk_src(c1), buf1, sem1).wait()
        process(c1, buf1)
        return 0

    lax.fori_loop(0, SPW // 2, gbody, 0)

    pltpu.sync_copy(out_v, out_hbm.at[pl.ds(wbase * AA, SPW * AA)])


_SC_GATHER_CACHE = []


def _sc_gather(*args):
    if not _SC_GATHER_CACHE:
        _SC_GATHER_CACHE.append(functools.partial(
            pl.kernel,
            out_type=jax.ShapeDtypeStruct((S * AA,), jnp.float32),
            mesh=plsc.VectorSubcoreMesh(core_axis_name="c",
                                        subcore_axis_name="s",
                                        num_cores=NC, num_subcores=NS),
            scratch_types=[
                pltpu.VMEM((K, BLK), jnp.float32),
                pltpu.VMEM((K, BLK), jnp.float32),
                pltpu.VMEM((SPW * K,), jnp.int32),
                pltpu.VMEM((S,), jnp.int32),
                pltpu.VMEM((SPW * AA,), jnp.float32),
                pltpu.SemaphoreType.DMA,
                pltpu.SemaphoreType.DMA,
            ],
            compiler_params=pltpu.CompilerParams(needs_layout_passes=False,
                                                 use_tc_tiling_on_sc=True),
        )(_sc_body))
    return _SC_GATHER_CACHE[0](*args)


def _eaa_body(eidx_hbm, ref_hbm, out_hbm, eidx_v, ref_v, out_v, sem):
    wid = lax.axis_index("s") * NC + lax.axis_index("c")
    wbase = wid * SPW
    b = wid // (L // SPW)
    l0 = (wid % (L // SPW)) * SPW
    b512 = b * L
    pltpu.sync_copy(eidx_hbm.at[pl.ds(wbase * K, SPW * K)], eidx_v)
    pltpu.sync_copy(ref_hbm, ref_v)

    lane = lax.iota(jnp.int32, LANES)

    def kbody(k, _):
        def gbody(g, _):
            sl = g * LANES + lane
            eidx = plsc.load_gather(eidx_v, [sl * K + k])
            out_v[k, pl.ds(g * LANES, LANES)] = plsc.load_gather(
                ref_v, [eidx + b512])
            return 0
        lax.fori_loop(0, SPW // LANES, gbody, 0)
        return 0

    lax.fori_loop(0, K, kbody, 0)
    pltpu.sync_copy(out_v, out_hbm.at[b, :, pl.ds(l0, SPW)])


_EAA_CACHE = []


def _sc_eaa(*args):
    if not _EAA_CACHE:
        _EAA_CACHE.append(functools.partial(
            pl.kernel,
            out_type=jax.ShapeDtypeStruct((B, K, L), jnp.int32),
            mesh=plsc.VectorSubcoreMesh(core_axis_name="c",
                                        subcore_axis_name="s",
                                        num_cores=NC, num_subcores=NS),
            scratch_types=[
                pltpu.VMEM((SPW * K,), jnp.int32),
                pltpu.VMEM((S,), jnp.int32),
                pltpu.VMEM((K, SPW), jnp.int32),
                pltpu.SemaphoreType.DMA,
            ],
            compiler_params=pltpu.CompilerParams(needs_layout_passes=False),
        )(_eaa_body))
    return _EAA_CACHE[0](*args)


def _tc_body(eaa_ref, etab_ref, out_ref):
    kk = pl.program_id(1)
    e = etab_ref[0, 0]
    eaa = eaa_ref[0, 0, 0]
    jrow = lax.broadcasted_iota(jnp.int32, (BLK, L), 0)
    jm = jrow % AA
    m_pair = (jm == eaa[None, :]).astype(jnp.float32)
    m_diag = (jm == jrow // AA).astype(jnp.float32)
    mask = jnp.where(kk == 0, m_diag, m_pair)
    seg = (lax.broadcasted_iota(jnp.int32, (AA, BLK), 1) // AA ==
           lax.broadcasted_iota(jnp.int32, (AA, BLK), 0)).astype(jnp.float32)
    contrib = jnp.dot(seg, mask * e, preferred_element_type=jnp.float32)

    @pl.when(kk == 0)
    def _():
        out_ref[...] = contrib[None]

    @pl.when(kk > 0)
    def _():
        out_ref[...] += contrib[None]


def _tc_gather(eaa, etab_t):
    return pl.pallas_call(
        _tc_body,
        grid=(B, K),
        in_specs=[
            pl.BlockSpec((1, 1, 1, L), lambda b, k: (b, k, 0, 0)),
            pl.BlockSpec((1, 1, BLK, L), lambda b, k: (b, k, 0, 0)),
        ],
        out_specs=pl.BlockSpec((1, AA, L), lambda b, k: (b, 0, 0)),
        out_shape=jax.ShapeDtypeStruct((B, AA, L), jnp.float32),
    )(eaa.reshape(B, K, 1, L), etab_t)


def _fin_body(aa_ref, ref_ref, mask_ref, out_ref):
    neg = -aa_ref[...]
    m = jnp.max(neg, axis=1, keepdims=True)
    lse = jnp.log(jnp.sum(jnp.exp(neg - m), axis=1)) + m[:, 0, :]
    r = ref_ref[...]
    sel = lax.broadcasted_iota(jnp.int32, (B, AA, L), 1) == r[:, None, :]
    picked = jnp.sum(jnp.where(sel, neg, 0.0), axis=1)
    mask = mask_ref[...]
    num = jnp.sum((picked - lse) * mask, axis=1, keepdims=True)
    den = jnp.sum(mask, axis=1, keepdims=True)
    out_ref[0, 0] = -jnp.sum(num / den) / B


_finish = pl.pallas_call(
    _fin_body,
    out_shape=jax.ShapeDtypeStruct((1, 1), jnp.float32),
    out_specs=pl.BlockSpec(memory_space=pltpu.SMEM),
)


def kernel(etab, E_idx, ref_seqs, x_mask):
    etab_t = jnp.transpose(etab, (0, 2, 3, 1))
    eaa = _sc_eaa(E_idx.reshape(-1), ref_seqs.reshape(-1))
    aa_nrgs = _tc_gather(eaa, etab_t)
    out = _finish(aa_nrgs, ref_seqs, x_mask)
    return out[0, 0]


# speedup vs baseline: 7.6375x; 2.3027x over previous
import functools

import jax
import jax.numpy as jnp
from jax import lax
from jax.experimental import pallas as pl
from jax.experimental.pallas import tpu as pltpu
from jax.experimental.pallas import tpu_sc as plsc

B, L, K, AA = 8, 512, 30, 20
S = B * L
BLK = AA * AA

NC, NS, LANES = 2, 16, 16
NW = NC * NS
SPW = S // NW
CH = 4
NCHUNK = SPW // CH
CHW = CH * K * BLK
BUF_W = CHW + 256


def _sc_body(etab_hbm, eidx_hbm, ref_hbm, out_hbm,
             buf0, buf1, eidx_v, ref_v, out_v, sem0, sem1):
    wid = lax.axis_index("s") * NC + lax.axis_index("c")
    wbase = wid * SPW
    b512 = (wid // (L // SPW)) * L

    pltpu.sync_copy(eidx_hbm.at[pl.ds(wbase * K, SPW * K)], eidx_v)
    pltpu.sync_copy(ref_hbm, ref_v)

    i_lo = lax.iota(jnp.int32, LANES)
    i_hi = i_lo + LANES
    hi_mask = i_lo < (AA - LANES)
    hi_sc = jnp.minimum(i_lo, AA - LANES - 1) + LANES

    def splat(x):
        return jnp.full((LANES,), x, jnp.int32)

    def chunk_src(c):
        return etab_hbm.at[wbase + c]

    def process(c, buf):
        site = c
        acc0 = plsc.load_gather(buf, [splat(0), i_lo * (AA + 1)])
        acc1 = plsc.load_gather(
            buf, [splat(0), jnp.minimum(i_hi * (AA + 1), BLK - 1)])

        def kbody(k, accs):
            a0, a1 = accs
            eidx = plsc.load_gather(eidx_v, [splat(site * K + k)])
            eaa = plsc.load_gather(ref_v, [eidx + b512])
            kk = splat(k)
            a0 = a0 + plsc.load_gather(buf, [kk, i_lo * AA + eaa])
            a1 = a1 + plsc.load_gather(
                buf, [kk, jnp.minimum(i_hi * AA + eaa, BLK - 1)])
            return a0, a1

        acc0, acc1 = lax.fori_loop(1, K, kbody, (acc0, acc1))
        ob = site * AA
        out_v[pl.ds(ob, LANES)] = acc0
        plsc.store_scatter(out_v, [ob + hi_sc], acc1, mask=hi_mask)

    pltpu.async_copy(chunk_src(0), buf0, sem0)

    def gbody(g, _):
        c0 = 2 * g
        c1 = c0 + 1
        pltpu.async_copy(chunk_src(c1), buf1, sem1)
        pltpu.make_async_copy(chunk_src(c0), buf0, sem0).wait()
        process(c0, buf0)

        @pl.when(c1 + 1 < SPW)
        def _():
            pltpu.async_copy(chunk_src(c1 + 1), buf0, sem0)

        pltpu.make_async_copy(chunk_src(c1), buf1, sem1).wait()
        process(c1, buf1)
        return 0

    lax.fori_loop(0, SPW // 2, gbody, 0)

    pltpu.sync_copy(out_v, out_hbm.at[pl.ds(wbase * AA, SPW * AA)])


_SC_GATHER_CACHE = []


def _sc_gather(*args):
    if not _SC_GATHER_CACHE:
        _SC_GATHER_CACHE.append(functools.partial(
            pl.kernel,
            out_type=jax.ShapeDtypeStruct((S * AA,), jnp.float32),
            mesh=plsc.VectorSubcoreMesh(core_axis_name="c",
                                        subcore_axis_name="s",
                                        num_cores=NC, num_subcores=NS),
            scratch_types=[
                pltpu.VMEM((K, BLK), jnp.float32),
                pltpu.VMEM((K, BLK), jnp.float32),
                pltpu.VMEM((SPW * K,), jnp.int32),
                pltpu.VMEM((S,), jnp.int32),
                pltpu.VMEM((SPW * AA,), jnp.float32),
                pltpu.SemaphoreType.DMA,
                pltpu.SemaphoreType.DMA,
            ],
            compiler_params=pltpu.CompilerParams(needs_layout_passes=False,
                                                 use_tc_tiling_on_sc=True),
        )(_sc_body))
    return _SC_GATHER_CACHE[0](*args)


def _eaa_body(eidx_hbm, ref_hbm, out_hbm, eidx_v, ref_v, out_v, sem):
    wid = lax.axis_index("s") * NC + lax.axis_index("c")
    wbase = wid * SPW
    b = wid // (L // SPW)
    l0 = (wid % (L // SPW)) * SPW
    b512 = b * L
    pltpu.sync_copy(eidx_hbm.at[pl.ds(wbase * K, SPW * K)], eidx_v)
    pltpu.sync_copy(ref_hbm, ref_v)

    lane = lax.iota(jnp.int32, LANES)

    def kbody(k, _):
        def gbody(g, _):
            sl = g * LANES + lane
            eidx = plsc.load_gather(eidx_v, [sl * K + k])
            out_v[k, pl.ds(g * LANES, LANES)] = plsc.load_gather(
                ref_v, [eidx + b512])
            return 0
        lax.fori_loop(0, SPW // LANES, gbody, 0)
        return 0

    lax.fori_loop(0, K, kbody, 0)
    pltpu.sync_copy(out_v, out_hbm.at[b, :, pl.ds(l0, SPW)])


_EAA_CACHE = []


def _sc_eaa(*args):
    if not _EAA_CACHE:
        _EAA_CACHE.append(functools.partial(
            pl.kernel,
            out_type=jax.ShapeDtypeStruct((B, K, L), jnp.int32),
            mesh=plsc.VectorSubcoreMesh(core_axis_name="c",
                                        subcore_axis_name="s",
                                        num_cores=NC, num_subcores=NS),
            scratch_types=[
                pltpu.VMEM((SPW * K,), jnp.int32),
                pltpu.VMEM((S,), jnp.int32),
                pltpu.VMEM((K, SPW), jnp.int32),
                pltpu.SemaphoreType.DMA,
            ],
            compiler_params=pltpu.CompilerParams(needs_layout_passes=False),
        )(_eaa_body))
    return _EAA_CACHE[0](*args)


KB = 10


def _tc_body(eaa_ref, etab_ref, out_ref):
    kk = pl.program_id(1)
    jrow = lax.broadcasted_iota(jnp.int32, (BLK, L), 0)
    jm = jrow % AA
    diag = jm == jrow // AA
    acc = None
    for dk in range(KB):
        e = etab_ref[0, dk]
        eaa = eaa_ref[0, dk, 0]
        sel = jnp.where(jm == eaa[None, :], e, 0.0)
        if dk == 0:
            sel = jnp.where(kk == 0, jnp.where(diag, e, 0.0), sel)
        acc = sel if dk == 0 else acc + sel
    seg = (lax.broadcasted_iota(jnp.int32, (AA, BLK), 1) // AA ==
           lax.broadcasted_iota(jnp.int32, (AA, BLK), 0)).astype(jnp.float32)
    contrib = jnp.dot(seg, acc, preferred_element_type=jnp.float32)

    @pl.when(kk == 0)
    def _():
        out_ref[...] = contrib[None]

    @pl.when(kk > 0)
    def _():
        out_ref[...] += contrib[None]


def _tc_gather(eaa, etab_t):
    return pl.pallas_call(
        _tc_body,
        grid=(B, K // KB),
        in_specs=[
            pl.BlockSpec((1, KB, 1, L), lambda b, k: (b, k, 0, 0)),
            pl.BlockSpec((1, KB, BLK, L), lambda b, k: (b, k, 0, 0)),
        ],
        out_specs=pl.BlockSpec((1, AA, L), lambda b, k: (b, 0, 0)),
        out_shape=jax.ShapeDtypeStruct((B, AA, L), jnp.float32),
    )(eaa.reshape(B, K, 1, L), etab_t)


def _fin_body(aa_ref, ref_ref, mask_ref, out_ref):
    neg = -aa_ref[...]
    m = jnp.max(neg, axis=1, keepdims=True)
    lse = jnp.log(jnp.sum(jnp.exp(neg - m), axis=1)) + m[:, 0, :]
    r = ref_ref[...]
    sel = lax.broadcasted_iota(jnp.int32, (B, AA, L), 1) == r[:, None, :]
    picked = jnp.sum(jnp.where(sel, neg, 0.0), axis=1)
    mask = mask_ref[...]
    num = jnp.sum((picked - lse) * mask, axis=1, keepdims=True)
    den = jnp.sum(mask, axis=1, keepdims=True)
    out_ref[0, 0] = -jnp.sum(num / den) / B


_finish = pl.pallas_call(
    _fin_body,
    out_shape=jax.ShapeDtypeStruct((1, 1), jnp.float32),
    out_specs=pl.BlockSpec(memory_space=pltpu.SMEM),
)


def kernel(etab, E_idx, ref_seqs, x_mask):
    etab_t = jnp.transpose(etab, (0, 2, 3, 1))
    eaa = _sc_eaa(E_idx.reshape(-1), ref_seqs.reshape(-1))
    aa_nrgs = _tc_gather(eaa, etab_t)
    out = _finish(aa_nrgs, ref_seqs, x_mask)
    return out[0, 0]


# speedup vs baseline: 7.8110x; 1.0227x over previous
import functools

import jax
import jax.numpy as jnp
from jax import lax
from jax.experimental import pallas as pl
from jax.experimental.pallas import tpu as pltpu
from jax.experimental.pallas import tpu_sc as plsc

B, L, K, AA = 8, 512, 30, 20
S = B * L
BLK = AA * AA

NC, NS, LANES = 2, 16, 16
NW = NC * NS
SPW = S // NW
CH = 4
NCHUNK = SPW // CH
CHW = CH * K * BLK
BUF_W = CHW + 256


def _sc_body(etab_hbm, eidx_hbm, ref_hbm, out_hbm,
             buf0, buf1, eidx_v, ref_v, out_v, sem0, sem1):
    wid = lax.axis_index("s") * NC + lax.axis_index("c")
    wbase = wid * SPW
    b512 = (wid // (L // SPW)) * L

    pltpu.sync_copy(eidx_hbm.at[pl.ds(wbase * K, SPW * K)], eidx_v)
    pltpu.sync_copy(ref_hbm, ref_v)

    i_lo = lax.iota(jnp.int32, LANES)
    i_hi = i_lo + LANES
    hi_mask = i_lo < (AA - LANES)
    hi_sc = jnp.minimum(i_lo, AA - LANES - 1) + LANES

    def splat(x):
        return jnp.full((LANES,), x, jnp.int32)

    def chunk_src(c):
        return etab_hbm.at[wbase + c]

    def process(c, buf):
        site = c
        acc0 = plsc.load_gather(buf, [splat(0), i_lo * (AA + 1)])
        acc1 = plsc.load_gather(
            buf, [splat(0), jnp.minimum(i_hi * (AA + 1), BLK - 1)])

        def kbody(k, accs):
            a0, a1 = accs
            eidx = plsc.load_gather(eidx_v, [splat(site * K + k)])
            eaa = plsc.load_gather(ref_v, [eidx + b512])
            kk = splat(k)
            a0 = a0 + plsc.load_gather(buf, [kk, i_lo * AA + eaa])
            a1 = a1 + plsc.load_gather(
                buf, [kk, jnp.minimum(i_hi * AA + eaa, BLK - 1)])
            return a0, a1

        acc0, acc1 = lax.fori_loop(1, K, kbody, (acc0, acc1))
        ob = site * AA
        out_v[pl.ds(ob, LANES)] = acc0
        plsc.store_scatter(out_v, [ob + hi_sc], acc1, mask=hi_mask)

    pltpu.async_copy(chunk_src(0), buf0, sem0)

    def gbody(g, _):
        c0 = 2 * g
        c1 = c0 + 1
        pltpu.async_copy(chunk_src(c1), buf1, sem1)
        pltpu.make_async_copy(chunk_src(c0), buf0, sem0).wait()
        process(c0, buf0)

        @pl.when(c1 + 1 < SPW)
        def _():
            pltpu.async_copy(chunk_src(c1 + 1), buf0, sem0)

        pltpu.make_async_copy(chunk_src(c1), buf1, sem1).wait()
        process(c1, buf1)
        return 0

    lax.fori_loop(0, SPW // 2, gbody, 0)

    pltpu.sync_copy(out_v, out_hbm.at[pl.ds(wbase * AA, SPW * AA)])


_SC_GATHER_CACHE = []


def _sc_gather(*args):
    if not _SC_GATHER_CACHE:
        _SC_GATHER_CACHE.append(functools.partial(
            pl.kernel,
            out_type=jax.ShapeDtypeStruct((S * AA,), jnp.float32),
            mesh=plsc.VectorSubcoreMesh(core_axis_name="c",
                                        subcore_axis_name="s",
                                        num_cores=NC, num_subcores=NS),
            scratch_types=[
                pltpu.VMEM((K, BLK), jnp.float32),
                pltpu.VMEM((K, BLK), jnp.float32),
                pltpu.VMEM((SPW * K,), jnp.int32),
                pltpu.VMEM((S,), jnp.int32),
                pltpu.VMEM((SPW * AA,), jnp.float32),
                pltpu.SemaphoreType.DMA,
                pltpu.SemaphoreType.DMA,
            ],
            compiler_params=pltpu.CompilerParams(needs_layout_passes=False,
                                                 use_tc_tiling_on_sc=True),
        )(_sc_body))
    return _SC_GATHER_CACHE[0](*args)


NW_E = NS
SPW_E = S // NW_E


def _eaa_body(eidx_hbm, ref_hbm, out_hbm, eidx_v, ref_v, out_v, sem):
    wid = lax.axis_index("s")
    wbase = wid * SPW_E
    b = wid // (L // SPW_E)
    l0 = (wid % (L // SPW_E)) * SPW_E
    b512 = b * L
    pltpu.sync_copy(eidx_hbm.at[pl.ds(wbase * K, SPW_E * K)], eidx_v)
    pltpu.sync_copy(ref_hbm, ref_v)

    lane = lax.iota(jnp.int32, LANES)

    def kbody(k, _):
        def gbody(g, _):
            sl = g * LANES + lane
            eidx = plsc.load_gather(eidx_v, [sl * K + k])
            out_v[k, pl.ds(g * LANES, LANES)] = plsc.load_gather(
                ref_v, [eidx + b512])
            return 0
        lax.fori_loop(0, SPW_E // LANES, gbody, 0)
        return 0

    lax.fori_loop(0, K, kbody, 0)
    pltpu.sync_copy(out_v, out_hbm.at[b, :, pl.ds(l0, SPW_E)])


_EAA_CACHE = []


def _sc_eaa(*args):
    if not _EAA_CACHE:
        _EAA_CACHE.append(functools.partial(
            pl.kernel,
            out_type=jax.ShapeDtypeStruct((B, K, L), jnp.int32),
            mesh=plsc.VectorSubcoreMesh(core_axis_name="c",
                                        subcore_axis_name="s",
                                        num_cores=1, num_subcores=NS),
            scratch_types=[
                pltpu.VMEM((SPW_E * K,), jnp.int32),
                pltpu.VMEM((S,), jnp.int32),
                pltpu.VMEM((K, SPW_E), jnp.int32),
                pltpu.SemaphoreType.DMA,
            ],
            compiler_params=pltpu.CompilerParams(needs_layout_passes=False),
        )(_eaa_body))
    return _EAA_CACHE[0](*args)


KB = 15


def _tc_body(eaa_ref, etab_ref, out_ref):
    kk = pl.program_id(1)
    jrow = lax.broadcasted_iota(jnp.int32, (BLK, L), 0)
    jm = jrow % AA
    diag = jm == jrow // AA
    acc = None
    for dk in range(KB):
        e = etab_ref[0, dk]
        eaa = eaa_ref[0, dk, 0]
        sel = jnp.where(jm == eaa[None, :], e, 0.0)
        if dk == 0:
            sel = jnp.where(kk == 0, jnp.where(diag, e, 0.0), sel)
        acc = sel if dk == 0 else acc + sel
    seg = (lax.broadcasted_iota(jnp.int32, (AA, BLK), 1) // AA ==
           lax.broadcasted_iota(jnp.int32, (AA, BLK), 0)).astype(jnp.float32)
    contrib = jnp.dot(seg, acc, preferred_element_type=jnp.float32)

    @pl.when(kk == 0)
    def _():
        out_ref[...] = contrib[None]

    @pl.when(kk > 0)
    def _():
        out_ref[...] += contrib[None]


def _tc_gather(eaa, etab_t):
    return pl.pallas_call(
        _tc_body,
        grid=(B, K // KB),
        in_specs=[
            pl.BlockSpec((1, KB, 1, L), lambda b, k: (b, k, 0, 0)),
            pl.BlockSpec((1, KB, BLK, L), lambda b, k: (b, k, 0, 0)),
        ],
        out_specs=pl.BlockSpec((1, AA, L), lambda b, k: (b, 0, 0)),
        out_shape=jax.ShapeDtypeStruct((B, AA, L), jnp.float32),
    )(eaa.reshape(B, K, 1, L), etab_t)


def _fin_body(aa_ref, ref_ref, mask_ref, out_ref):
    neg = -aa_ref[...]
    m = jnp.max(neg, axis=1, keepdims=True)
    lse = jnp.log(jnp.sum(jnp.exp(neg - m), axis=1)) + m[:, 0, :]
    r = ref_ref[...]
    sel = lax.broadcasted_iota(jnp.int32, (B, AA, L), 1) == r[:, None, :]
    picked = jnp.sum(jnp.where(sel, neg, 0.0), axis=1)
    mask = mask_ref[...]
    num = jnp.sum((picked - lse) * mask, axis=1, keepdims=True)
    den = jnp.sum(mask, axis=1, keepdims=True)
    out_ref[0, 0] = -jnp.sum(num / den) / B


_finish = pl.pallas_call(
    _fin_body,
    out_shape=jax.ShapeDtypeStruct((1, 1), jnp.float32),
    out_specs=pl.BlockSpec(memory_space=pltpu.SMEM),
)


def kernel(etab, E_idx, ref_seqs, x_mask):
    etab_t = jnp.transpose(etab, (0, 2, 3, 1))
    eaa = _sc_eaa(E_idx.reshape(-1), ref_seqs.reshape(-1))
    aa_nrgs = _tc_gather(eaa, etab_t)
    out = _finish(aa_nrgs, ref_seqs, x_mask)
    return out[0, 0]
